# baseline (speedup 1.0000x reference)
import functools

import jax
import jax.numpy as jnp
from jax import lax
from jax.experimental import pallas as pl
from jax.experimental.pallas import tpu as pltpu
from jax.experimental.pallas import tpu_sc as plsc

NC = 2
NS = 16
NW = NC * NS
L = 16

BATCH_SIZE = 16384
N_FACT = 16
B_PER_W = BATCH_SIZE // NW
IDX_CHUNK = 128
N_CHUNKS = B_PER_W // IDX_CHUNK


def _body(uidx_hbm, iidx_hbm, ufac_hbm, ifac_hbm, out_hbm,
          uidx_v, iidx_v, urows_v, irows_v, out_v, sem):
    wid = lax.axis_index("s") * NC + lax.axis_index("c")
    base = wid * B_PER_W

    pltpu.sync_copy(uidx_hbm.at[pl.ds(base, B_PER_W)], uidx_v)
    pltpu.sync_copy(iidx_hbm.at[pl.ds(base, B_PER_W)], iidx_v)

    copies = []
    for c in range(N_CHUNKS):
        sl = pl.ds(c * IDX_CHUNK, IDX_CHUNK)
        copies.append(pltpu.async_copy(
            ufac_hbm.at[uidx_v.at[sl]], urows_v.at[sl], sem))
        copies.append(pltpu.async_copy(
            ifac_hbm.at[iidx_v.at[sl]], irows_v.at[sl], sem))
    for cp in copies:
        cp.wait()

    lane = lax.iota(jnp.int32, L)

    def one_group(g, _):
        rows = g * L + lane
        acc = jnp.zeros((L,), jnp.float32)
        for f in range(N_FACT):
            col = jnp.full((L,), f, jnp.int32)
            uf = plsc.load_gather(urows_v, [rows, col])
            vf = plsc.load_gather(irows_v, [rows, col])
            acc = acc + uf * vf
        out_v[pl.ds(g * L, L)] = acc * 5.0
        return 0

    lax.fori_loop(0, B_PER_W // L, one_group, 0)
    pltpu.sync_copy(out_v, out_hbm.at[pl.ds(base, B_PER_W)])


@jax.jit
def _mf_kernel(u_idx, i_idx, user_factors, item_factors):
    mesh = plsc.VectorSubcoreMesh(core_axis_name="c", subcore_axis_name="s")
    return pl.kernel(
        _body,
        out_type=jax.ShapeDtypeStruct((BATCH_SIZE,), jnp.float32),
        mesh=mesh,
        compiler_params=pltpu.CompilerParams(
            use_tc_tiling_on_sc=False, needs_layout_passes=False),
        scratch_types=[
            pltpu.VMEM((B_PER_W,), jnp.int32),
            pltpu.VMEM((B_PER_W,), jnp.int32),
            pltpu.VMEM((B_PER_W, N_FACT), jnp.float32),
            pltpu.VMEM((B_PER_W, N_FACT), jnp.float32),
            pltpu.VMEM((B_PER_W,), jnp.float32),
            pltpu.SemaphoreType.DMA,
        ],
    )(u_idx, i_idx, user_factors, item_factors)


def kernel(user, item, user_factors, item_factors):
    return _mf_kernel(user - 1, item - 1, user_factors, item_factors)


# speedup vs baseline: 11.9588x; 11.9588x over previous
import jax
import jax.numpy as jnp
from jax import lax
from jax.experimental import pallas as pl
from jax.experimental.pallas import tpu as pltpu
from jax.experimental.pallas import tpu_sc as plsc

NC = 2
NS = 16
NW = NC * NS
L = 16

BATCH_SIZE = 16384
N_ROWS = 1000000
N_FACT = 16
B_PER_W = BATCH_SIZE // NW
N_GROUPS = B_PER_W // L


def _body(uidx_hbm, iidx_hbm, ufacT_hbm, ifacT_hbm, out_hbm,
          uidx_v, iidx_v, ublk_v, iblk_v, out_v, usem, isem):
    wid = lax.axis_index("s") * NC + lax.axis_index("c")
    base = wid * B_PER_W

    pltpu.sync_copy(uidx_hbm.at[pl.ds(base, B_PER_W)], uidx_v)
    pltpu.sync_copy(iidx_hbm.at[pl.ds(base, B_PER_W)], iidx_v)

    lane = lax.iota(jnp.int32, L)

    def one_group(g, _):
        uv = uidx_v[pl.ds(g * L, L)]
        iv = iidx_v[pl.ds(g * L, L)]
        ub = uv >> 4
        ib = iv >> 4
        copies = []
        for b in range(L):
            for tf in range(2):
                copies.append(pltpu.async_copy(
                    ufacT_hbm.at[tf, :, pl.ds(ub[b] * 16, 16)],
                    ublk_v.at[b, tf, :, pl.ds(0, 16)], usem))
                copies.append(pltpu.async_copy(
                    ifacT_hbm.at[tf, :, pl.ds(ib[b] * 16, 16)],
                    iblk_v.at[b, tf, :, pl.ds(0, 16)], isem))
        for cp in copies:
            cp.wait()

        uoff = uv & 15
        ioff = iv & 15
        acc = jnp.zeros((L,), jnp.float32)
        for k in range(N_FACT):
            tfv = jnp.full((L,), k // 8, jnp.int32)
            f8v = jnp.full((L,), k % 8, jnp.int32)
            uf = plsc.load_gather(ublk_v, [lane, tfv, f8v, uoff])
            vf = plsc.load_gather(iblk_v, [lane, tfv, f8v, ioff])
            acc = acc + uf * vf
        out_v[pl.ds(g * L, L)] = acc * 5.0
        return 0

    lax.fori_loop(0, N_GROUPS, one_group, 0)
    pltpu.sync_copy(out_v, out_hbm.at[pl.ds(base, B_PER_W)])


@jax.jit
def _mf_kernel(u_idx, i_idx, ufacT, ifacT):
    mesh = plsc.VectorSubcoreMesh(core_axis_name="c", subcore_axis_name="s")
    return pl.kernel(
        _body,
        out_type=jax.ShapeDtypeStruct((BATCH_SIZE,), jnp.float32),
        mesh=mesh,
        compiler_params=pltpu.CompilerParams(needs_layout_passes=False),
        scratch_types=[
            pltpu.VMEM((B_PER_W,), jnp.int32),
            pltpu.VMEM((B_PER_W,), jnp.int32),
            pltpu.VMEM((L, 2, 8, 128), jnp.float32),
            pltpu.VMEM((L, 2, 8, 128), jnp.float32),
            pltpu.VMEM((B_PER_W,), jnp.float32),
            pltpu.SemaphoreType.DMA,
            pltpu.SemaphoreType.DMA,
        ],
    )(u_idx, i_idx, ufacT, ifacT)


def kernel(user, item, user_factors, item_factors):
    ufT = user_factors.T.reshape(2, 8, N_ROWS)
    ifT = item_factors.T.reshape(2, 8, N_ROWS)
    return _mf_kernel(user - 1, item - 1, ufT, ifT)


# speedup vs baseline: 13.4249x; 1.1226x over previous
import jax
import jax.numpy as jnp
from jax import lax
from jax.experimental import pallas as pl
from jax.experimental.pallas import tpu as pltpu
from jax.experimental.pallas import tpu_sc as plsc

NC = 2
NS = 16
NW = NC * NS
L = 16

BATCH_SIZE = 16384
N_ROWS = 1000000
N_FACT = 16
B_PER_W = BATCH_SIZE // NW
N_GROUPS = B_PER_W // L


def _body(uidx_hbm, iidx_hbm, ufacT_hbm, ifacT_hbm, out_hbm,
          uidx_v, iidx_v, blk_v, out_v, usem, isem):
    wid = lax.axis_index("s") * NC + lax.axis_index("c")
    base = wid * B_PER_W

    pltpu.sync_copy(uidx_hbm.at[pl.ds(base, B_PER_W)], uidx_v)
    pltpu.sync_copy(iidx_hbm.at[pl.ds(base, B_PER_W)], iidx_v)

    lane = lax.iota(jnp.int32, L)

    def fire_group(g, buf):
        uv = uidx_v[pl.ds(g * L, L)]
        iv = iidx_v[pl.ds(g * L, L)]
        ub = uv >> 4
        ib = iv >> 4
        for b in range(L):
            pltpu.async_copy(
                ufacT_hbm.at[:, :, pl.ds(ub[b] * 16, 16)],
                blk_v.at[buf, b, :, :, pl.ds(0, 16)], usem)
            pltpu.async_copy(
                ifacT_hbm.at[:, :, pl.ds(ib[b] * 16, 16)],
                blk_v.at[buf, b, :, :, pl.ds(16, 16)], isem)

    def drain_group():
        for b in range(L):
            pltpu.make_async_copy(
                ufacT_hbm.at[:, :, pl.ds(0, 16)],
                blk_v.at[0, b, :, :, pl.ds(0, 16)], usem).wait()
            pltpu.make_async_copy(
                ifacT_hbm.at[:, :, pl.ds(0, 16)],
                blk_v.at[0, b, :, :, pl.ds(16, 16)], isem).wait()

    def compute_group(g, buf):
        uv = uidx_v[pl.ds(g * L, L)]
        iv = iidx_v[pl.ds(g * L, L)]
        uoff = uv & 15
        ioff = (iv & 15) + 16
        bufv = jnp.full((L,), buf, jnp.int32)
        acc = jnp.zeros((L,), jnp.float32)
        for k in range(N_FACT):
            tfv = jnp.full((L,), k // 8, jnp.int32)
            f8v = jnp.full((L,), k % 8, jnp.int32)
            uf = plsc.load_gather(blk_v, [bufv, lane, tfv, f8v, uoff])
            vf = plsc.load_gather(blk_v, [bufv, lane, tfv, f8v, ioff])
            acc = acc + uf * vf
        out_v[pl.ds(g * L, L)] = acc * 5.0

    fire_group(0, 0)

    def one_group(g, _):
        buf = lax.rem(g, 2)

        @pl.when(g < N_GROUPS - 1)
        def _():
            fire_group(g + 1, 1 - buf)

        drain_group()
        compute_group(g, buf)
        return 0

    lax.fori_loop(0, N_GROUPS, one_group, 0)
    pltpu.sync_copy(out_v, out_hbm.at[pl.ds(base, B_PER_W)])


@jax.jit
def _mf_kernel(u_idx, i_idx, ufacT, ifacT):
    mesh = plsc.VectorSubcoreMesh(core_axis_name="c", subcore_axis_name="s")
    return pl.kernel(
        _body,
        out_type=jax.ShapeDtypeStruct((BATCH_SIZE,), jnp.float32),
        mesh=mesh,
        compiler_params=pltpu.CompilerParams(needs_layout_passes=False),
        scratch_types=[
            pltpu.VMEM((B_PER_W,), jnp.int32),
            pltpu.VMEM((B_PER_W,), jnp.int32),
            pltpu.VMEM((2, L, 2, 8, 128), jnp.float32),
            pltpu.VMEM((B_PER_W,), jnp.float32),
            pltpu.SemaphoreType.DMA,
            pltpu.SemaphoreType.DMA,
        ],
    )(u_idx, i_idx, ufacT, ifacT)


def kernel(user, item, user_factors, item_factors):
    ufT = user_factors.T.reshape(2, 8, N_ROWS)
    ifT = item_factors.T.reshape(2, 8, N_ROWS)
    return _mf_kernel(user - 1, item - 1, ufT, ifT)
